# baseline (speedup 1.0000x reference)
import functools

import jax
import jax.numpy as jnp
from jax import lax
from jax.experimental import pallas as pl
from jax.experimental.pallas import tpu as pltpu
from jax.experimental.pallas import tpu_sc as plsc

_NUM_FIELDS = 26
_FIELD_SIZE = 38462
_BATCH = 16384
_NC, _NS, _LANES = 2, 16, 16
_NW = _NC * _NS
_RPW = _BATCH // _NW
_IPW = _RPW * _NUM_FIELDS
_CHUNK = 128
_NCHUNK = _IPW // _CHUNK

_mesh = plsc.VectorSubcoreMesh(
    core_axis_name="c", subcore_axis_name="s",
    num_cores=_NC, num_subcores=_NS,
)


@functools.partial(
    pl.kernel,
    out_type=jax.ShapeDtypeStruct((1, _BATCH), jnp.float32),
    mesh=_mesh,
    scratch_types=[
        pltpu.VMEM((_NUM_FIELDS, _RPW), jnp.int32),
        pltpu.VMEM((_NCHUNK, _CHUNK), jnp.int32),
        pltpu.VMEM((_NUM_FIELDS, _RPW), jnp.float32),
        pltpu.VMEM((_RPW,), jnp.float32),
        pltpu.VMEM((_LANES,), jnp.float32),
        pltpu.SemaphoreType.DMA,
    ],
)
def _features_linear_sc(x_hbm, table_hbm, bias_hbm, out_hbm,
                        xw, idxw, valw, acc, biasw, gsem):
    wid = lax.axis_index("s") * _NC + lax.axis_index("c")
    table1d = table_hbm.at[0]

    pltpu.sync_copy(x_hbm.at[:, pl.ds(wid * _RPW, _RPW)], xw)
    pltpu.sync_copy(bias_hbm, biasw.at[pl.ds(0, 1)])

    def _gfire(c, carry):
        f = c >> 2
        col = (c & 3) * _CHUNK
        offv = jnp.zeros((_LANES,), jnp.int32) + f * _FIELD_SIZE
        for t in range(_CHUNK // _LANES):
            sl = pl.ds(t * _LANES, _LANES)
            idxw[c, sl] = xw[f, pl.ds(col + t * _LANES, _LANES)] + offv
        pltpu.make_async_copy(table1d.at[idxw.at[c]],
                              valw.at[f, pl.ds(col, _CHUNK)], gsem).start()
        return carry
    lax.fori_loop(0, _NCHUNK, _gfire, 0, unroll=2)

    def _gdrain(c, carry):
        f = c >> 2
        col = (c & 3) * _CHUNK
        pltpu.make_async_copy(table1d.at[idxw.at[c]],
                              valw.at[f, pl.ds(col, _CHUNK)], gsem).wait()
        return carry
    lax.fori_loop(0, _NCHUNK, _gdrain, 0, unroll=4)

    bias_vec = jnp.zeros((_LANES,), jnp.float32) + biasw[...][0]

    def _red_body(j, carry):
        sl = pl.ds(j * _LANES, _LANES)
        v0 = bias_vec
        v1 = valw[1, sl]
        for f in range(2, _NUM_FIELDS, 2):
            v0 = v0 + valw[f, sl]
            v1 = v1 + valw[f + 1, sl]
        acc[sl] = (v0 + valw[0, sl]) + v1
        return carry
    lax.fori_loop(0, _RPW // _LANES, _red_body, 0, unroll=2)

    pltpu.sync_copy(acc, out_hbm.at[0, pl.ds(wid * _RPW, _RPW)])


def kernel(x, table, bias):
    x_t = x.T
    table_t = table.T

    out = _features_linear_sc(x_t, table_t, bias.astype(jnp.float32))
    return out.reshape(_BATCH, 1)


# speedup vs baseline: 1.2214x; 1.2214x over previous
import functools

import jax
import jax.numpy as jnp
from jax import lax
from jax.experimental import pallas as pl
from jax.experimental.pallas import tpu as pltpu
from jax.experimental.pallas import tpu_sc as plsc

_NUM_FIELDS = 26
_FIELD_SIZE = 38462
_BATCH = 16384
_NC, _NS, _LANES = 2, 16, 16
_NW = _NC * _NS
_RPW = _BATCH // _NW
_IPW = _RPW * _NUM_FIELDS
_CHUNK = 128
_NCHUNK = _IPW // _CHUNK
_VOCAB = _NUM_FIELDS * _FIELD_SIZE
_VPAD = (_VOCAB + 127) // 128 * 128
_VSLICE = (_VPAD // _NS + 127) // 128 * 128

_mesh = plsc.VectorSubcoreMesh(
    core_axis_name="c", subcore_axis_name="s",
    num_cores=_NC, num_subcores=_NS,
)


@functools.partial(
    pl.kernel,
    out_type=jax.ShapeDtypeStruct((1, _BATCH), jnp.float32),
    mesh=_mesh,
    scratch_types=[
        pltpu.VMEM((_NUM_FIELDS, _RPW), jnp.int32),
        pltpu.VMEM((_NCHUNK, _CHUNK), jnp.int32),
        pltpu.VMEM((_NUM_FIELDS, _RPW), jnp.float32),
        pltpu.VMEM((_RPW,), jnp.float32),
        pltpu.VMEM((_LANES,), jnp.float32),
        pltpu.VMEM_SHARED((_VPAD,), jnp.float32),
        pltpu.SemaphoreType.DMA,
    ],
)
def _features_linear_sc(x_hbm, table_hbm, bias_hbm, out_hbm,
                        xw, idxw, valw, acc, biasw, table_sh, gsem):
    wid = lax.axis_index("s") * _NC + lax.axis_index("c")
    sid = lax.axis_index("s")

    pltpu.sync_copy(x_hbm.at[:, pl.ds(wid * _RPW, _RPW)], xw)
    pltpu.sync_copy(bias_hbm, biasw.at[pl.ds(0, 1)])
    base = pl.multiple_of(
        jnp.minimum(sid * _VSLICE, _VPAD - _VSLICE), 128)
    pltpu.sync_copy(table_hbm.at[0, pl.ds(base, _VSLICE)],
                    table_sh.at[pl.ds(base, _VSLICE)])
    plsc.subcore_barrier()
    table1d = table_sh

    def _gfire(c, carry):
        f = c >> 2
        col = (c & 3) * _CHUNK
        offv = jnp.zeros((_LANES,), jnp.int32) + f * _FIELD_SIZE
        for t in range(_CHUNK // _LANES):
            sl = pl.ds(t * _LANES, _LANES)
            idxw[c, sl] = xw[f, pl.ds(col + t * _LANES, _LANES)] + offv
        pltpu.make_async_copy(table1d.at[idxw.at[c]],
                              valw.at[f, pl.ds(col, _CHUNK)], gsem).start()
        return carry
    lax.fori_loop(0, _NCHUNK, _gfire, 0)

    def _gdrain(c, carry):
        f = c >> 2
        col = (c & 3) * _CHUNK
        pltpu.make_async_copy(table1d.at[idxw.at[c]],
                              valw.at[f, pl.ds(col, _CHUNK)], gsem).wait()
        return carry
    lax.fori_loop(0, _NCHUNK, _gdrain, 0)

    bias_vec = jnp.zeros((_LANES,), jnp.float32) + biasw[...][0]

    def _red_body(j, carry):
        sl = pl.ds(j * _LANES, _LANES)
        v0 = bias_vec
        v1 = valw[1, sl]
        for f in range(2, _NUM_FIELDS, 2):
            v0 = v0 + valw[f, sl]
            v1 = v1 + valw[f + 1, sl]
        acc[sl] = (v0 + valw[0, sl]) + v1
        return carry
    lax.fori_loop(0, _RPW // _LANES, _red_body, 0)

    pltpu.sync_copy(acc, out_hbm.at[0, pl.ds(wid * _RPW, _RPW)])


def kernel(x, table, bias):
    x_t = x.T
    table_t = table.T

    out = _features_linear_sc(x_t, table_t, bias.astype(jnp.float32))
    return out.reshape(_BATCH, 1)


# speedup vs baseline: 1.2682x; 1.0383x over previous
import functools

import jax
import jax.numpy as jnp
from jax import lax
from jax.experimental import pallas as pl
from jax.experimental.pallas import tpu as pltpu
from jax.experimental.pallas import tpu_sc as plsc

_NUM_FIELDS = 26
_FIELD_SIZE = 38462
_BATCH = 16384
_NC, _NS, _LANES = 2, 16, 16
_NW = _NC * _NS
_RPW = _BATCH // _NW
_IPW = _RPW * _NUM_FIELDS
_CHUNK = 128
_NCHUNK = _IPW // _CHUNK
_VOCAB = _NUM_FIELDS * _FIELD_SIZE
_VPAD = (_VOCAB + 127) // 128 * 128
_VSLICE = (_VPAD // _NS + 127) // 128 * 128

_mesh = plsc.VectorSubcoreMesh(
    core_axis_name="c", subcore_axis_name="s",
    num_cores=_NC, num_subcores=_NS,
)


@functools.partial(
    pl.kernel,
    out_type=jax.ShapeDtypeStruct((1, _BATCH), jnp.float32),
    mesh=_mesh,
    scratch_types=[
        pltpu.VMEM((_NUM_FIELDS, _RPW), jnp.int32),
        pltpu.VMEM((_NCHUNK, _CHUNK), jnp.int32),
        pltpu.VMEM((_NUM_FIELDS, _RPW), jnp.float32),
        pltpu.VMEM((_RPW,), jnp.float32),
        pltpu.VMEM((_LANES,), jnp.float32),
        pltpu.VMEM_SHARED((_VPAD,), jnp.float32),
        pltpu.SemaphoreType.DMA,
        pltpu.SemaphoreType.DMA,
    ],
)
def _features_linear_sc(x_hbm, table_hbm, bias_hbm, out_hbm,
                        xw, idxw, valw, acc, biasw, table_sh, gsem, tsem):
    wid = lax.axis_index("s") * _NC + lax.axis_index("c")
    sid = lax.axis_index("s")

    base = pl.multiple_of(
        jnp.minimum(sid * _VSLICE, _VPAD - _VSLICE), 128)
    tcopy = pltpu.make_async_copy(table_hbm.at[0, pl.ds(base, _VSLICE)],
                                  table_sh.at[pl.ds(base, _VSLICE)], tsem)
    tcopy.start()
    pltpu.sync_copy(x_hbm.at[:, pl.ds(wid * _RPW, _RPW)], xw)
    pltpu.sync_copy(bias_hbm, biasw.at[pl.ds(0, 1)])
    table1d = table_sh

    def _idx_body(c, carry):
        f = c >> 2
        col = (c & 3) * _CHUNK
        offv = jnp.zeros((_LANES,), jnp.int32) + f * _FIELD_SIZE
        for t in range(_CHUNK // _LANES):
            sl = pl.ds(t * _LANES, _LANES)
            idxw[c, sl] = xw[f, pl.ds(col + t * _LANES, _LANES)] + offv
        return carry
    lax.fori_loop(0, _NCHUNK, _idx_body, 0)

    tcopy.wait()
    plsc.subcore_barrier()

    def _gfire(c, carry):
        f = c >> 2
        col = (c & 3) * _CHUNK
        pltpu.make_async_copy(table1d.at[idxw.at[c]],
                              valw.at[f, pl.ds(col, _CHUNK)], gsem).start()
        return carry
    lax.fori_loop(0, _NCHUNK, _gfire, 0)

    def _gdrain(c, carry):
        f = c >> 2
        col = (c & 3) * _CHUNK
        pltpu.make_async_copy(table1d.at[idxw.at[c]],
                              valw.at[f, pl.ds(col, _CHUNK)], gsem).wait()
        return carry
    lax.fori_loop(0, _NCHUNK, _gdrain, 0)

    bias_vec = jnp.zeros((_LANES,), jnp.float32) + biasw[...][0]

    def _red_body(j, carry):
        sl = pl.ds(j * _LANES, _LANES)
        v0 = bias_vec
        v1 = valw[1, sl]
        for f in range(2, _NUM_FIELDS, 2):
            v0 = v0 + valw[f, sl]
            v1 = v1 + valw[f + 1, sl]
        acc[sl] = (v0 + valw[0, sl]) + v1
        return carry
    lax.fori_loop(0, _RPW // _LANES, _red_body, 0)

    pltpu.sync_copy(acc, out_hbm.at[0, pl.ds(wid * _RPW, _RPW)])


def kernel(x, table, bias):
    x_t = x.T
    table_t = table.T

    out = _features_linear_sc(x_t, table_t, bias.astype(jnp.float32))
    return out.reshape(_BATCH, 1)
